# Initial kernel scaffold; baseline (speedup 1.0000x reference)
import jax
import jax.numpy as jnp
from jax.experimental import pallas as pl


def kernel(x_0, x_1, x_2, adjacency_0, incidence_2, incidence_1_t, proj0_w, proj0_b, proj1_w, proj1_b, proj2_w, proj2_b, l0_w11, l0_w21, l0_w01, l0_uw, l0_ub, l1_w11, l1_w21, l1_w01, l1_uw, l1_ub):
    raise NotImplementedError("write your pallas kernel here")


# speedup vs baseline: 1.2652x; 1.2652x over previous
import functools

import jax
import jax.numpy as jnp
from jax.experimental import pallas as pl
from jax.experimental.pallas import tpu as pltpu

N_EDGES = 8192
N_NODES = 8192
N_FACES = 2048
HID = 32
ROW_BLK = 512


def _elu(x):
    return jnp.where(x > 0, x, jnp.exp(x) - 1.0)


def _proj_body(x0_ref, x1_ref, x2_ref, w0_ref, b0_ref, w1_ref, b1_ref,
               w2_ref, b2_ref, x0p_ref, x1p_ref, x2p_ref):
    x0p_ref[...] = _elu(
        jnp.dot(x0_ref[...], w0_ref[...], preferred_element_type=jnp.float32)
        + b0_ref[...])
    x1p_ref[...] = _elu(
        jnp.dot(x1_ref[...], w1_ref[...], preferred_element_type=jnp.float32)
        + b1_ref[...])
    x2p_ref[...] = _elu(
        jnp.dot(x2_ref[...], w2_ref[...], preferred_element_type=jnp.float32)
        + b2_ref[...])


def _statics_body(i1t_ref, i2_ref, x0p_ref, x2p_ref, w01a_ref, w01b_ref,
                  w21a_ref, w21b_ref, st0_ref, st1_ref):
    p0 = jnp.dot(i1t_ref[...], x0p_ref[...], preferred_element_type=jnp.float32)
    p2 = jnp.dot(i2_ref[...], x2p_ref[...], preferred_element_type=jnp.float32)
    st0_ref[...] = (
        _elu(jnp.dot(p0, w01a_ref[...], preferred_element_type=jnp.float32))
        + _elu(jnp.dot(p2, w21a_ref[...], preferred_element_type=jnp.float32)))
    st1_ref[...] = (
        _elu(jnp.dot(p0, w01b_ref[...], preferred_element_type=jnp.float32))
        + _elu(jnp.dot(p2, w21b_ref[...], preferred_element_type=jnp.float32)))


def _layer_body(adj_ref, x1_ref, static_ref, w11_ref, uw_ref, ub_ref,
                out_ref, y1_scr):
    @pl.when(pl.program_id(0) == 0)
    def _():
        y1_scr[...] = jnp.dot(x1_ref[...], w11_ref[...],
                              preferred_element_type=jnp.float32)

    x_up = _elu(jnp.dot(adj_ref[...], y1_scr[...],
                        preferred_element_type=jnp.float32))
    agg = x_up + static_ref[...]
    out_ref[...] = _elu(
        jnp.dot(agg, uw_ref[...], preferred_element_type=jnp.float32)
        + ub_ref[...])


@jax.jit
def kernel(x_0, x_1, x_2, adjacency_0, incidence_2, incidence_1_t,
           proj0_w, proj0_b, proj1_w, proj1_b, proj2_w, proj2_b,
           l0_w11, l0_w21, l0_w01, l0_uw, l0_ub,
           l1_w11, l1_w21, l1_w01, l1_uw, l1_ub):
    f32 = jnp.float32
    b0 = proj0_b.reshape(1, HID)
    b1 = proj1_b.reshape(1, HID)
    b2 = proj2_b.reshape(1, HID)
    ub0 = l0_ub.reshape(1, HID)
    ub1 = l1_ub.reshape(1, HID)

    x0p, x1p, x2p = pl.pallas_call(
        _proj_body,
        out_shape=(
            jax.ShapeDtypeStruct((N_NODES, HID), f32),
            jax.ShapeDtypeStruct((N_EDGES, HID), f32),
            jax.ShapeDtypeStruct((N_FACES, HID), f32),
        ),
    )(x_0, x_1, x_2, proj0_w, b0, proj1_w, b1, proj2_w, b2)

    n_blocks = N_EDGES // ROW_BLK
    const2 = lambda i: (0, 0)
    static0, static1 = pl.pallas_call(
        _statics_body,
        grid=(n_blocks,),
        in_specs=[
            pl.BlockSpec((ROW_BLK, N_NODES), lambda i: (i, 0)),
            pl.BlockSpec((ROW_BLK, N_FACES), lambda i: (i, 0)),
            pl.BlockSpec((N_NODES, HID), const2),
            pl.BlockSpec((N_FACES, HID), const2),
            pl.BlockSpec((HID, HID), const2),
            pl.BlockSpec((HID, HID), const2),
            pl.BlockSpec((HID, HID), const2),
            pl.BlockSpec((HID, HID), const2),
        ],
        out_specs=(
            pl.BlockSpec((ROW_BLK, HID), lambda i: (i, 0)),
            pl.BlockSpec((ROW_BLK, HID), lambda i: (i, 0)),
        ),
        out_shape=(
            jax.ShapeDtypeStruct((N_EDGES, HID), f32),
            jax.ShapeDtypeStruct((N_EDGES, HID), f32),
        ),
        compiler_params=pltpu.CompilerParams(
            dimension_semantics=("arbitrary",)),
    )(incidence_1_t, incidence_2, x0p, x2p, l0_w01, l1_w01, l0_w21, l1_w21)

    layer_call = pl.pallas_call(
        _layer_body,
        grid=(n_blocks,),
        in_specs=[
            pl.BlockSpec((ROW_BLK, N_EDGES), lambda i: (i, 0)),
            pl.BlockSpec((N_EDGES, HID), const2),
            pl.BlockSpec((ROW_BLK, HID), lambda i: (i, 0)),
            pl.BlockSpec((HID, HID), const2),
            pl.BlockSpec((HID, HID), const2),
            pl.BlockSpec((1, HID), const2),
        ],
        out_specs=pl.BlockSpec((ROW_BLK, HID), lambda i: (i, 0)),
        out_shape=jax.ShapeDtypeStruct((N_EDGES, HID), f32),
        scratch_shapes=[pltpu.VMEM((N_EDGES, HID), f32)],
        compiler_params=pltpu.CompilerParams(
            dimension_semantics=("arbitrary",)),
    )

    x1_l0 = layer_call(adjacency_0, x1p, static0, l0_w11, l0_uw, ub0)
    x1_l1 = layer_call(adjacency_0, x1_l0, static1, l1_w11, l1_uw, ub1)

    return (x0p, x1_l1, x2p)


# speedup vs baseline: 1.2904x; 1.0199x over previous
import functools

import jax
import jax.numpy as jnp
from jax.experimental import pallas as pl
from jax.experimental.pallas import tpu as pltpu

N_EDGES = 8192
N_NODES = 8192
N_FACES = 2048
HID = 32
ROW_BLK = 256


def _elu(x):
    return jnp.where(x > 0, x, jnp.exp(x) - 1.0)


def _proj_body(x0_ref, x1_ref, x2_ref, w0_ref, b0_ref, w1_ref, b1_ref,
               w2_ref, b2_ref, x0p_ref, x1p_ref, x2p_ref):
    x0p_ref[...] = _elu(
        jnp.dot(x0_ref[...], w0_ref[...], preferred_element_type=jnp.float32)
        + b0_ref[...])
    x1p_ref[...] = _elu(
        jnp.dot(x1_ref[...], w1_ref[...], preferred_element_type=jnp.float32)
        + b1_ref[...])
    x2p_ref[...] = _elu(
        jnp.dot(x2_ref[...], w2_ref[...], preferred_element_type=jnp.float32)
        + b2_ref[...])


def _statics_body(i1t_ref, i2_ref, x0p_ref, x2p_ref, w01a_ref, w01b_ref,
                  w21a_ref, w21b_ref, st0_ref, st1_ref):
    p0 = jnp.dot(i1t_ref[...], x0p_ref[...], preferred_element_type=jnp.float32)
    p2 = jnp.dot(i2_ref[...], x2p_ref[...], preferred_element_type=jnp.float32)
    st0_ref[...] = (
        _elu(jnp.dot(p0, w01a_ref[...], preferred_element_type=jnp.float32))
        + _elu(jnp.dot(p2, w21a_ref[...], preferred_element_type=jnp.float32)))
    st1_ref[...] = (
        _elu(jnp.dot(p0, w01b_ref[...], preferred_element_type=jnp.float32))
        + _elu(jnp.dot(p2, w21b_ref[...], preferred_element_type=jnp.float32)))


def _layer_body(adj_ref, x1_ref, static_ref, w11_ref, uw_ref, ub_ref,
                out_ref, y1_scr):
    @pl.when(pl.program_id(0) == 0)
    def _():
        y1_scr[...] = jnp.dot(x1_ref[...], w11_ref[...],
                              preferred_element_type=jnp.float32)

    x_up = _elu(jnp.dot(adj_ref[...], y1_scr[...],
                        preferred_element_type=jnp.float32))
    agg = x_up + static_ref[...]
    out_ref[...] = _elu(
        jnp.dot(agg, uw_ref[...], preferred_element_type=jnp.float32)
        + ub_ref[...])


@jax.jit
def kernel(x_0, x_1, x_2, adjacency_0, incidence_2, incidence_1_t,
           proj0_w, proj0_b, proj1_w, proj1_b, proj2_w, proj2_b,
           l0_w11, l0_w21, l0_w01, l0_uw, l0_ub,
           l1_w11, l1_w21, l1_w01, l1_uw, l1_ub):
    f32 = jnp.float32
    b0 = proj0_b.reshape(1, HID)
    b1 = proj1_b.reshape(1, HID)
    b2 = proj2_b.reshape(1, HID)
    ub0 = l0_ub.reshape(1, HID)
    ub1 = l1_ub.reshape(1, HID)

    x0p, x1p, x2p = pl.pallas_call(
        _proj_body,
        out_shape=(
            jax.ShapeDtypeStruct((N_NODES, HID), f32),
            jax.ShapeDtypeStruct((N_EDGES, HID), f32),
            jax.ShapeDtypeStruct((N_FACES, HID), f32),
        ),
    )(x_0, x_1, x_2, proj0_w, b0, proj1_w, b1, proj2_w, b2)

    n_blocks = N_EDGES // ROW_BLK
    const2 = lambda i: (0, 0)
    static0, static1 = pl.pallas_call(
        _statics_body,
        grid=(n_blocks,),
        in_specs=[
            pl.BlockSpec((ROW_BLK, N_NODES), lambda i: (i, 0)),
            pl.BlockSpec((ROW_BLK, N_FACES), lambda i: (i, 0)),
            pl.BlockSpec((N_NODES, HID), const2),
            pl.BlockSpec((N_FACES, HID), const2),
            pl.BlockSpec((HID, HID), const2),
            pl.BlockSpec((HID, HID), const2),
            pl.BlockSpec((HID, HID), const2),
            pl.BlockSpec((HID, HID), const2),
        ],
        out_specs=(
            pl.BlockSpec((ROW_BLK, HID), lambda i: (i, 0)),
            pl.BlockSpec((ROW_BLK, HID), lambda i: (i, 0)),
        ),
        out_shape=(
            jax.ShapeDtypeStruct((N_EDGES, HID), f32),
            jax.ShapeDtypeStruct((N_EDGES, HID), f32),
        ),
        compiler_params=pltpu.CompilerParams(
            dimension_semantics=("arbitrary",)),
    )(incidence_1_t, incidence_2, x0p, x2p, l0_w01, l1_w01, l0_w21, l1_w21)

    layer_call = pl.pallas_call(
        _layer_body,
        grid=(n_blocks,),
        in_specs=[
            pl.BlockSpec((ROW_BLK, N_EDGES), lambda i: (i, 0)),
            pl.BlockSpec((N_EDGES, HID), const2),
            pl.BlockSpec((ROW_BLK, HID), lambda i: (i, 0)),
            pl.BlockSpec((HID, HID), const2),
            pl.BlockSpec((HID, HID), const2),
            pl.BlockSpec((1, HID), const2),
        ],
        out_specs=pl.BlockSpec((ROW_BLK, HID), lambda i: (i, 0)),
        out_shape=jax.ShapeDtypeStruct((N_EDGES, HID), f32),
        scratch_shapes=[pltpu.VMEM((N_EDGES, HID), f32)],
        compiler_params=pltpu.CompilerParams(
            dimension_semantics=("arbitrary",)),
    )

    x1_l0 = layer_call(adjacency_0, x1p, static0, l0_w11, l0_uw, ub0)
    x1_l1 = layer_call(adjacency_0, x1_l0, static1, l1_w11, l1_uw, ub1)

    return (x0p, x1_l1, x2p)
